# Initial kernel scaffold; baseline (speedup 1.0000x reference)
import jax
import jax.numpy as jnp
from jax.experimental import pallas as pl


def kernel(x, edge_index, batch, global_features, Wl, bl, Wr, ln_g, ln_b, m0_W, m0_b, m0_g, m0_bb, mb_W, mb_b, mb_g, mb_bb, fin_W, fin_b):
    raise NotImplementedError("write your pallas kernel here")


# speedup vs baseline: 6.4195x; 6.4195x over previous
import functools

import jax
import jax.numpy as jnp
from jax import lax
from jax.experimental import pallas as pl
from jax.experimental.pallas import tpu as pltpu
from jax.experimental.pallas import tpu_sc as plsc

N = 10000
E = 320000
D = 128
G = 16
GF = 24
L = 7
LC = 32

NC = 2
NS = 16
NW = NC * NS
EPW = E // NW
CHUNK = 80
KCH = EPW // CHUNK
N_PAD = 10240
RPT = N_PAD // NS
DEGW = 128

@functools.cache
def _get_sc_agg():
    return pl.kernel(
        _sc_agg_body,
        out_type=jax.ShapeDtypeStruct((NC, N_PAD, D), jnp.float32),
        mesh=plsc.VectorSubcoreMesh(core_axis_name="c", subcore_axis_name="s"),
        scratch_types=[
            pltpu.VMEM((KCH, CHUNK), jnp.int32),
            pltpu.VMEM((KCH, CHUNK), jnp.int32),
            pltpu.VMEM((CHUNK, D), jnp.float32),
            pltpu.SemaphoreType.DMA,
            pltpu.VMEM_SHARED((N_PAD, D), jnp.float32),
        ],
    )


def _sc_agg_body(h_hbm, src_hbm, dst_hbm, zeros_hbm, out_hbm,
                 src_v, dst_v, rows_v, sem, agg_sh):
    c = lax.axis_index("c")
    s = lax.axis_index("s")
    wid = s * NC + c
    r0 = s * RPT
    pltpu.sync_copy(zeros_hbm.at[pl.ds(r0, RPT)], agg_sh.at[pl.ds(r0, RPT)])
    pltpu.sync_copy(src_hbm.at[wid], src_v)
    pltpu.sync_copy(dst_hbm.at[wid], dst_v)
    plsc.subcore_barrier()

    def body(j, _):
        pltpu.async_copy(h_hbm.at[src_v.at[j]], rows_v, sem).wait()
        pltpu.sync_copy(rows_v, agg_sh.at[dst_v.at[j]], add=True)
        return ()

    lax.fori_loop(0, KCH, body, ())
    plsc.subcore_barrier()
    pltpu.sync_copy(agg_sh.at[pl.ds(r0, RPT)], out_hbm.at[c, pl.ds(r0, RPT)])


@functools.cache
def _get_sc_deg():
    return pl.kernel(
        _sc_deg_body,
        out_type=jax.ShapeDtypeStruct((NC, N_PAD, DEGW), jnp.float32),
        mesh=plsc.VectorSubcoreMesh(core_axis_name="c", subcore_axis_name="s"),
        scratch_types=[
            pltpu.VMEM((KCH, CHUNK), jnp.int32),
            pltpu.VMEM((CHUNK, DEGW), jnp.float32),
            pltpu.VMEM_SHARED((N_PAD, DEGW), jnp.float32),
        ],
    )


def _sc_deg_body(dst_hbm, zeros_hbm, ones_hbm, out_hbm, dst_v, ones_v, deg_sh):
    c = lax.axis_index("c")
    s = lax.axis_index("s")
    wid = s * NC + c
    r0 = s * RPT
    pltpu.sync_copy(zeros_hbm.at[pl.ds(r0, RPT)], deg_sh.at[pl.ds(r0, RPT)])
    pltpu.sync_copy(dst_hbm.at[wid], dst_v)
    pltpu.sync_copy(ones_hbm, ones_v)
    plsc.subcore_barrier()

    def body(j, _):
        pltpu.sync_copy(ones_v, deg_sh.at[dst_v.at[j]], add=True)
        return ()

    lax.fori_loop(0, KCH, body, ())
    plsc.subcore_barrier()
    pltpu.sync_copy(deg_sh.at[pl.ds(r0, RPT)], out_hbm.at[c, pl.ds(r0, RPT)])


_SQRT_HALF = 0.7071067811865476


def _gelu(x):
    return 0.5 * x * (1.0 + lax.erf(x * _SQRT_HALF))


def _ln(x, g, b, eps=1e-5):
    mu = jnp.mean(x, axis=-1, keepdims=True)
    var = jnp.mean((x - mu) ** 2, axis=-1, keepdims=True)
    return (x - mu) * lax.rsqrt(var + eps) * g + b


_RTC = 1000
_NBLK = N // _RTC


def _layer_tc_body(add_res, p_ref, h_ref, rd_ref, wl_ref, bl_ref, wr_ref,
                   g_ref, b_ref, o_ref):
    h = h_ref[...]
    agg = (p_ref[0] + p_ref[1]) * rd_ref[...]
    f = (jnp.dot(agg, wl_ref[...], preferred_element_type=jnp.float32)
         + jnp.dot(h, wr_ref[...], preferred_element_type=jnp.float32)
         + bl_ref[...])
    f = _ln(_gelu(f), g_ref[...], b_ref[...])
    if add_res:
        f = f + h
    o_ref[...] = f


def _tc_rdeg(degp):
    def body(dg_ref, o_ref):
        o_ref[...] = 1.0 / jnp.maximum(dg_ref[0, :, 0:1] + dg_ref[1, :, 0:1],
                                       1.0)
    return pl.pallas_call(
        body,
        grid=(_NBLK,),
        in_specs=[pl.BlockSpec((NC, _RTC, DEGW), lambda i: (0, i, 0))],
        out_specs=pl.BlockSpec((_RTC, 1), lambda i: (i, 0)),
        out_shape=jax.ShapeDtypeStruct((N, 1), jnp.float32),
    )(degp)


def _tc_layer(add_res, p, h, rdeg, wl, bl, wr, g, b):
    return pl.pallas_call(
        functools.partial(_layer_tc_body, add_res),
        grid=(_NBLK,),
        in_specs=[
            pl.BlockSpec((NC, _RTC, D), lambda i: (0, i, 0)),
            pl.BlockSpec((_RTC, D), lambda i: (i, 0)),
            pl.BlockSpec((_RTC, 1), lambda i: (i, 0)),
            pl.BlockSpec((D, D), lambda i: (0, 0)),
            pl.BlockSpec((1, D), lambda i: (0, 0)),
            pl.BlockSpec((D, D), lambda i: (0, 0)),
            pl.BlockSpec((1, D), lambda i: (0, 0)),
            pl.BlockSpec((1, D), lambda i: (0, 0)),
        ],
        out_specs=pl.BlockSpec((_RTC, D), lambda i: (i, 0)),
        out_shape=jax.ShapeDtypeStruct((N, D), jnp.float32),
    )(p, h, rdeg, wl, bl, wr, g, b)


def _poolhead_body(h_ref, bt_ref, gfp_ref, wa_ref, wb_ref, m0b_ref, m0g_ref,
                   m0bb_ref, mbw_ref, mbb_ref, mbg_ref, mbbb_ref, finw_ref,
                   finb_ref, o_ref, pool_acc, cnt_acc):
    i = pl.program_id(0)

    @pl.when(i == 0)
    def _():
        pool_acc[...] = jnp.zeros_like(pool_acc)
        cnt_acc[...] = jnp.zeros_like(cnt_acc)

    h = h_ref[...]
    bt = bt_ref[...]
    gid = lax.broadcasted_iota(jnp.int32, (_RTC, G), 1)
    oh = (bt == gid).astype(jnp.float32)
    dn = (((0,), (0,)), ((), ()))
    pool_acc[...] += lax.dot_general(oh, h, dn,
                                     preferred_element_type=jnp.float32)
    cnt_acc[...] += lax.dot_general(oh, jnp.ones_like(h), dn,
                                    preferred_element_type=jnp.float32)

    @pl.when(i == _NBLK - 1)
    def _():
        pooled = pool_acc[...] / jnp.maximum(cnt_acc[...], 1.0)
        z = (jnp.dot(pooled, wa_ref[...], preferred_element_type=jnp.float32)
             + jnp.dot(gfp_ref[...], wb_ref[...],
                       preferred_element_type=jnp.float32)
             + m0b_ref[...])
        z = _ln(_gelu(z), m0g_ref[...], m0bb_ref[...])
        for j in range(3):
            f = jnp.dot(z, mbw_ref[j], preferred_element_type=jnp.float32)
            f = _ln(_gelu(f + mbb_ref[j:j + 1, :]), mbg_ref[j:j + 1, :],
                    mbbb_ref[j:j + 1, :])
            z = f + z
        o_ref[...] = (jnp.dot(z, finw_ref[...],
                              preferred_element_type=jnp.float32)
                      + finb_ref[...])


def _tc_poolhead(h, batchf, gf, wa, wb, m0b, m0g, m0bb, mbw, mbb, mbg, mbbb,
                 finw, finb):
    full = lambda shape: pl.BlockSpec(shape, lambda i: tuple(0 for _ in shape))
    return pl.pallas_call(
        _poolhead_body,
        grid=(_NBLK,),
        in_specs=[
            pl.BlockSpec((_RTC, D), lambda i: (i, 0)),
            pl.BlockSpec((_RTC, 1), lambda i: (i, 0)),
            full((G, GF)),
            full((D, LC)),
            full((GF, LC)),
            full((1, LC)),
            full((1, LC)),
            full((1, LC)),
            full((3, LC, LC)),
            full((3, LC)),
            full((3, LC)),
            full((3, LC)),
            full((LC, 1)),
            full((1, 1)),
        ],
        out_specs=full((G, 1)),
        out_shape=jax.ShapeDtypeStruct((G, 1), jnp.float32),
        scratch_shapes=[
            pltpu.VMEM((G, D), jnp.float32),
            pltpu.VMEM((G, D), jnp.float32),
        ],
    )(h, batchf, gf, wa, wb, m0b, m0g, m0bb, mbw, mbb, mbg, mbbb, finw, finb)


def kernel(x, edge_index, batch, global_features, Wl, bl, Wr, ln_g, ln_b,
           m0_W, m0_b, m0_g, m0_bb, mb_W, mb_b, mb_g, mb_bb, fin_W, fin_b):
    src = edge_index[0].reshape(NW, KCH, CHUNK)
    dst = edge_index[1].reshape(NW, KCH, CHUNK)
    zeros_nd = jnp.zeros((N_PAD, D), jnp.float32)
    ones_cd = jnp.ones((CHUNK, DEGW), jnp.float32)

    degp = _get_sc_deg()(dst, zeros_nd, ones_cd)
    rdeg = _tc_rdeg(degp)

    h = x
    for i in range(L):
        p = _get_sc_agg()(h, src, dst, zeros_nd)
        h = _tc_layer(i > 0, p, h, rdeg, Wl[i], bl[i].reshape(1, D), Wr[i],
                      ln_g[i].reshape(1, D), ln_b[i].reshape(1, D))

    out = _tc_poolhead(
        h, batch.reshape(N, 1), global_features,
        m0_W[:D], m0_W[D:], m0_b.reshape(1, LC), m0_g.reshape(1, LC),
        m0_bb.reshape(1, LC), mb_W, mb_b, mb_g, mb_bb, fin_W,
        fin_b.reshape(1, 1))
    return out
